# baseline (speedup 1.0000x reference)
import functools

import jax
import jax.numpy as jnp
from jax import lax
from jax.experimental import pallas as pl
from jax.experimental.pallas import tpu as pltpu
from jax.experimental.pallas import tpu_sc as plsc

N = 10000
E = 320000
F_IN = 128
EMB = 16
H = 143
HP = 144
G = 64
GF = 32
PH = 128
NTYPES = 400

NW = 32
EPW = E // NW
CHUNK = 50
NCHUNK = EPW // CHUNK
IB = 20
NBUF = 5
NB = NCHUNK // IB
NSUB = 16
RPT = N // NSUB

ROWS_BLK = 2000


def _h0_body(x_ref, s_ref, embp_ref, o_ref):
    x = x_ref[...]
    nt = x[:, 0:1].astype(jnp.int32)
    iota = lax.broadcasted_iota(jnp.int32, (1, NTYPES), 1)
    oh = (nt == iota).astype(jnp.float32)
    o_ref[...] = (
        jnp.dot(x, s_ref[...], preferred_element_type=jnp.float32)
        + jnp.dot(oh, embp_ref[...], preferred_element_type=jnp.float32)
    )


def _mlp_body(h_ref, p_ref, w1_ref, b1_ref, w2_ref, b2_ref, g_ref, bb_ref, o_ref):
    h = h_ref[...]
    z = h + p_ref[0] + p_ref[1]
    a = jnp.maximum(
        jnp.dot(z, w1_ref[...], preferred_element_type=jnp.float32) + b1_ref[...], 0.0
    )
    z2 = jnp.dot(a, w2_ref[...], preferred_element_type=jnp.float32) + b2_ref[...]
    mu = jnp.sum(z2, axis=1, keepdims=True) * (1.0 / H)
    var = jnp.sum(z2 * z2, axis=1, keepdims=True) * (1.0 / H) - mu * mu
    zn = (z2 - mu) * lax.rsqrt(var + 1e-5) * g_ref[...] + bb_ref[...]
    o_ref[...] = h + jnp.maximum(zn, 0.0)


def _pool_body(h_ref, bcol_ref, brow_ref, gf_ref, wg1_ref, bg1_ref, wg2_ref,
               bg2_ref, wca_ref, wcb_ref, bc1_ref, wc2_ref, bc2_ref, o_ref):
    h = h_ref[...]
    a = jnp.maximum(
        jnp.dot(h, wg1_ref[...], preferred_element_type=jnp.float32) + bg1_ref[...], 0.0
    )
    gate = jnp.sum(a * wg2_ref[...], axis=1, keepdims=True) + bg2_ref[...]
    bcol = bcol_ref[...]
    ohb = bcol == lax.broadcasted_iota(jnp.int32, (1, G), 1)
    ohf = ohb.astype(jnp.float32)
    gmax = jnp.max(jnp.where(ohb, gate, -3e38), axis=0, keepdims=True)
    gmax_pn = jnp.sum(ohf * gmax, axis=1, keepdims=True)
    e = jnp.exp(gate - gmax_pn)
    denom = jnp.sum(ohf * e, axis=0, keepdims=True)
    denom_pn = jnp.sum(ohf * denom, axis=1, keepdims=True)
    alpha = e / (denom_pn + 1e-16)
    oht = (brow_ref[...] == lax.broadcasted_iota(jnp.int32, (G, 1), 0)).astype(
        jnp.float32
    )
    pooled = jnp.dot(oht, alpha * h, preferred_element_type=jnp.float32)
    c1 = jnp.maximum(
        jnp.dot(pooled, wca_ref[...], preferred_element_type=jnp.float32)
        + jnp.dot(gf_ref[...], wcb_ref[...], preferred_element_type=jnp.float32)
        + bc1_ref[...],
        0.0,
    )
    o_ref[...] = jnp.dot(c1, wc2_ref[...], preferred_element_type=jnp.float32) + bc2_ref[...]


def _mlp_pool_body(h_ref, p_ref, w1_ref, b1_ref, w2_ref, b2_ref, g_ref, bb_ref,
                   bcol_ref, brow_ref, gf_ref, wg1_ref, bg1_ref, wg2_ref,
                   bg2_ref, wca_ref, wcb_ref, bc1_ref, wc2_ref, bc2_ref,
                   o_ref, h2_ref):
    i = pl.program_id(0)

    @pl.when(i < N // ROWS_BLK)
    def _():
        h = h_ref[...]
        z = h + p_ref[0] + p_ref[1]
        a = jnp.maximum(
            jnp.dot(z, w1_ref[...], preferred_element_type=jnp.float32)
            + b1_ref[...], 0.0
        )
        z2 = jnp.dot(a, w2_ref[...], preferred_element_type=jnp.float32) + b2_ref[...]
        mu = jnp.sum(z2, axis=1, keepdims=True) * (1.0 / H)
        var = jnp.sum(z2 * z2, axis=1, keepdims=True) * (1.0 / H) - mu * mu
        zn = (z2 - mu) * lax.rsqrt(var + 1e-5) * g_ref[...] + bb_ref[...]
        h2_ref[pl.ds(i * ROWS_BLK, ROWS_BLK), :] = h + jnp.maximum(zn, 0.0)

    @pl.when(i == N // ROWS_BLK)
    def _():
        _pool_body(h2_ref, bcol_ref, brow_ref, gf_ref, wg1_ref, bg1_ref,
                   wg2_ref, bg2_ref, wca_ref, wcb_ref, bc1_ref, wc2_ref,
                   bc2_ref, o_ref)


def _full(shape):
    return pl.BlockSpec(shape, lambda *_: tuple(0 for _ in shape))


def _h0_call(x, s, embp):
    grid = (N // ROWS_BLK,)
    return pl.pallas_call(
        _h0_body,
        grid=grid,
        in_specs=[
            pl.BlockSpec((ROWS_BLK, F_IN), lambda i: (i, 0)),
            _full((F_IN, HP)),
            _full((NTYPES, HP)),
        ],
        out_specs=pl.BlockSpec((ROWS_BLK, HP), lambda i: (i, 0)),
        out_shape=jax.ShapeDtypeStruct((N, HP), jnp.float32),
    )(x, s, embp)


def _mlp_call(h, p, w1, b1, w2, b2, g, bb):
    grid = (N // ROWS_BLK,)
    return pl.pallas_call(
        _mlp_body,
        grid=grid,
        in_specs=[
            pl.BlockSpec((ROWS_BLK, HP), lambda i: (i, 0)),
            pl.BlockSpec((2, ROWS_BLK, HP), lambda i: (0, i, 0)),
            _full((HP, HP)),
            _full((1, HP)),
            _full((HP, HP)),
            _full((1, HP)),
            _full((1, HP)),
            _full((1, HP)),
        ],
        out_specs=pl.BlockSpec((ROWS_BLK, HP), lambda i: (i, 0)),
        out_shape=jax.ShapeDtypeStruct((N, HP), jnp.float32),
    )(h, p, w1, b1, w2, b2, g, bb)


def _mlp_pool_call(h, p, w1, b1, w2, b2, g, bb,
                   bcol, brow, gf, wg1, bg1, wg2, bg2, wca, wcb, bc1, wc2, bc2):
    nblk = N // ROWS_BLK
    clamp = lambda i: (jnp.minimum(i, nblk - 1), 0)
    clamp3 = lambda i: (0, jnp.minimum(i, nblk - 1), 0)
    return pl.pallas_call(
        _mlp_pool_body,
        grid=(nblk + 1,),
        in_specs=[
            pl.BlockSpec((ROWS_BLK, HP), clamp),
            pl.BlockSpec((2, ROWS_BLK, HP), clamp3),
            _full((HP, HP)),
            _full((1, HP)),
            _full((HP, HP)),
            _full((1, HP)),
            _full((1, HP)),
            _full((1, HP)),
            _full((N, 1)),
            _full((1, N)),
            _full((G, GF)),
            _full((HP, PH)),
            _full((1, PH)),
            _full((1, PH)),
            _full((1, 1)),
            _full((HP, PH)),
            _full((GF, PH)),
            _full((1, PH)),
            _full((PH, 2)),
            _full((1, 2)),
        ],
        out_specs=_full((G, 2)),
        out_shape=jax.ShapeDtypeStruct((G, 2), jnp.float32),
        scratch_shapes=[pltpu.VMEM((N, HP), jnp.float32)],
    )(h, p, w1, b1, w2, b2, g, bb, bcol, brow, gf, wg1, bg1, wg2, bg2,
      wca, wcb, bc1, wc2, bc2)


def _sc_agg_body(h_hbm, src_hbm, dst_hbm, zeros_hbm, out_hbm,
                 src_v, dst_v, rows0, rows1, rows2, rows3, rows4, acc_sh,
                 sem0, sem1, sem2, sem3, sem4):
    rows = (rows0, rows1, rows2, rows3, rows4)
    sems = (sem0, sem1, sem2, sem3, sem4)
    c = lax.axis_index("c")
    s = lax.axis_index("s")
    w = c * NSUB + s
    pltpu.sync_copy(zeros_hbm, acc_sh.at[pl.ds(s * RPT, RPT)])
    plsc.subcore_barrier()

    def block(bb, carry):
        base = w * NCHUNK + bb * IB
        pltpu.sync_copy(src_hbm.at[pl.ds(base, IB)], src_v)
        pltpu.sync_copy(dst_hbm.at[pl.ds(base, IB)], dst_v)
        for b in range(NBUF - 1):
            pltpu.async_copy(h_hbm.at[src_v.at[b]], rows[b], sems[b])

        def group(jj, carry2):
            j0 = jj * NBUF
            for b in range(NBUF):
                j = j0 + b
                pltpu.make_async_copy(
                    h_hbm.at[src_v.at[j]], rows[b], sems[b]
                ).wait()
                nxt = j + NBUF - 1
                bn = (b + NBUF - 1) % NBUF

                @pl.when(nxt < IB)
                def _():
                    pltpu.async_copy(
                        h_hbm.at[src_v.at[nxt]], rows[bn], sems[bn]
                    )

                pltpu.sync_copy(rows[b], acc_sh.at[dst_v.at[j]], add=True)
            return carry2

        lax.fori_loop(0, IB // NBUF, group, 0)
        return carry

    lax.fori_loop(0, NB, block, 0)
    plsc.subcore_barrier()
    pltpu.sync_copy(
        acc_sh.at[pl.ds(s * RPT, RPT)],
        out_hbm.at[pl.ds(c * N + s * RPT, RPT)],
    )


@functools.cache
def _sc_agg_kernel():
    mesh = plsc.VectorSubcoreMesh(core_axis_name="c", subcore_axis_name="s")
    return pl.kernel(
        _sc_agg_body,
        out_type=jax.ShapeDtypeStruct((2 * N, HP), jnp.float32),
        mesh=mesh,
        compiler_params=pltpu.CompilerParams(use_tc_tiling_on_sc=False),
        scratch_types=[
            pltpu.VMEM((IB, CHUNK), jnp.int32),
            pltpu.VMEM((IB, CHUNK), jnp.int32),
            pltpu.VMEM((CHUNK, HP), jnp.float32),
            pltpu.VMEM((CHUNK, HP), jnp.float32),
            pltpu.VMEM((CHUNK, HP), jnp.float32),
            pltpu.VMEM((CHUNK, HP), jnp.float32),
            pltpu.VMEM((CHUNK, HP), jnp.float32),
            pltpu.VMEM_SHARED((N, HP), jnp.float32),
            pltpu.SemaphoreType.DMA,
            pltpu.SemaphoreType.DMA,
            pltpu.SemaphoreType.DMA,
            pltpu.SemaphoreType.DMA,
            pltpu.SemaphoreType.DMA,
        ],
    )


def kernel(x, edge_index, batch, global_features, emb, W1a, b1a, W2a, b2a,
           W1b, b1b, W2b, b2b, ln1g, ln1b, ln2g, ln2b, Wg1, bg1, Wg2, bg2,
           Wc1, bc1, Wc2, bc2):
    f32 = jnp.float32

    s_mat = jnp.eye(F_IN, HP, k=-1, dtype=f32)
    embp = jnp.pad(emb, ((0, 0), (F_IN - 1, HP - F_IN + 1 - EMB)))

    def pad2(wm):
        return jnp.pad(wm, ((0, HP - H), (0, HP - H)))

    def padr(v):
        return jnp.pad(v, (0, HP - H)).reshape(1, HP)

    w1a, w2a, w1b, w2b = pad2(W1a), pad2(W2a), pad2(W1b), pad2(W2b)
    b1a_, b2a_, b1b_, b2b_ = padr(b1a), padr(b2a), padr(b1b), padr(b2b)
    g1, bb1, g2, bb2 = padr(ln1g), padr(ln1b), padr(ln2g), padr(ln2b)
    wg1 = jnp.pad(Wg1, ((0, HP - H), (0, 0)))
    wg2 = Wg2.reshape(1, PH)
    bg1_ = bg1.reshape(1, PH)
    bg2_ = bg2.reshape(1, 1)
    wca = jnp.pad(Wc1[:H], ((0, HP - H), (0, 0)))
    wcb = Wc1[H:]
    bc1_ = bc1.reshape(1, PH)
    bc2_ = bc2.reshape(1, 2)

    src3 = edge_index[0].reshape(NW * NCHUNK, CHUNK)
    dst3 = edge_index[1].reshape(NW * NCHUNK, CHUNK)
    zeros_chunk = jnp.zeros((RPT, HP), f32)
    bcol = batch.reshape(N, 1)
    brow = batch.reshape(1, N)

    sc_agg = _sc_agg_kernel()

    h0 = _h0_call(x, s_mat, embp)
    p1 = sc_agg(h0, src3, dst3, zeros_chunk).reshape(2, N, HP)
    h1 = _mlp_call(h0, p1, w1a, b1a_, w2a, b2a_, g1, bb1)
    p2 = sc_agg(h1, src3, dst3, zeros_chunk).reshape(2, N, HP)
    return _mlp_pool_call(h1, p2, w1b, b1b_, w2b, b2b_, g2, bb2,
                          bcol, brow, global_features, wg1, bg1_, wg2, bg2_,
                          wca, wcb, bc1_, Wc2, bc2_)


# speedup vs baseline: 1.0611x; 1.0611x over previous
import functools

import jax
import jax.numpy as jnp
from jax import lax
from jax.experimental import pallas as pl
from jax.experimental.pallas import tpu as pltpu
from jax.experimental.pallas import tpu_sc as plsc

N = 10000
E = 320000
F_IN = 128
EMB = 16
H = 143
HP = 144
G = 64
GF = 32
PH = 128
NTYPES = 400

NW = 32
EPW = E // NW
CHUNK = 50
NCHUNK = EPW // CHUNK
IB = 40
NBUF = 4
NB = NCHUNK // IB
NSUB = 16
RPT = N // NSUB

ROWS_BLK = 2000


def _h0_body(x_ref, s_ref, embp_ref, o_ref):
    x = x_ref[...]
    nt = x[:, 0:1].astype(jnp.int32)
    iota = lax.broadcasted_iota(jnp.int32, (1, NTYPES), 1)
    oh = (nt == iota).astype(jnp.float32)
    o_ref[...] = (
        jnp.dot(x, s_ref[...], preferred_element_type=jnp.float32)
        + jnp.dot(oh, embp_ref[...], preferred_element_type=jnp.float32)
    )


def _mlp_body(h_ref, p_ref, w1_ref, b1_ref, w2_ref, b2_ref, g_ref, bb_ref, o_ref):
    h = h_ref[...]
    z = h + p_ref[0] + p_ref[1]
    a = jnp.maximum(
        jnp.dot(z, w1_ref[...], preferred_element_type=jnp.float32) + b1_ref[...], 0.0
    )
    z2 = jnp.dot(a, w2_ref[...], preferred_element_type=jnp.float32) + b2_ref[...]
    mu = jnp.sum(z2, axis=1, keepdims=True) * (1.0 / H)
    var = jnp.sum(z2 * z2, axis=1, keepdims=True) * (1.0 / H) - mu * mu
    zn = (z2 - mu) * lax.rsqrt(var + 1e-5) * g_ref[...] + bb_ref[...]
    o_ref[...] = h + jnp.maximum(zn, 0.0)


def _pool_body(h_ref, bcol_ref, brow_ref, gf_ref, wg1_ref, bg1_ref, wg2_ref,
               bg2_ref, wca_ref, wcb_ref, bc1_ref, wc2_ref, bc2_ref, o_ref):
    h = h_ref[...]
    a = jnp.maximum(
        jnp.dot(h, wg1_ref[...], preferred_element_type=jnp.float32) + bg1_ref[...], 0.0
    )
    gate = jnp.sum(a * wg2_ref[...], axis=1, keepdims=True) + bg2_ref[...]
    bcol = bcol_ref[...]
    ohb = bcol == lax.broadcasted_iota(jnp.int32, (1, G), 1)
    ohf = ohb.astype(jnp.float32)
    gmax = jnp.max(jnp.where(ohb, gate, -3e38), axis=0, keepdims=True)
    gmax_pn = jnp.sum(ohf * gmax, axis=1, keepdims=True)
    e = jnp.exp(gate - gmax_pn)
    denom = jnp.sum(ohf * e, axis=0, keepdims=True)
    denom_pn = jnp.sum(ohf * denom, axis=1, keepdims=True)
    alpha = e / (denom_pn + 1e-16)
    oht = (brow_ref[...] == lax.broadcasted_iota(jnp.int32, (G, 1), 0)).astype(
        jnp.float32
    )
    pooled = jnp.dot(oht, alpha * h, preferred_element_type=jnp.float32)
    c1 = jnp.maximum(
        jnp.dot(pooled, wca_ref[...], preferred_element_type=jnp.float32)
        + jnp.dot(gf_ref[...], wcb_ref[...], preferred_element_type=jnp.float32)
        + bc1_ref[...],
        0.0,
    )
    o_ref[...] = jnp.dot(c1, wc2_ref[...], preferred_element_type=jnp.float32) + bc2_ref[...]


def _mlp_pool_body(h_ref, p_ref, w1_ref, b1_ref, w2_ref, b2_ref, g_ref, bb_ref,
                   bcol_ref, brow_ref, gf_ref, wg1_ref, bg1_ref, wg2_ref,
                   bg2_ref, wca_ref, wcb_ref, bc1_ref, wc2_ref, bc2_ref,
                   o_ref, h2_ref):
    i = pl.program_id(0)

    @pl.when(i < N // ROWS_BLK)
    def _():
        h = h_ref[...]
        z = h + p_ref[0] + p_ref[1]
        a = jnp.maximum(
            jnp.dot(z, w1_ref[...], preferred_element_type=jnp.float32)
            + b1_ref[...], 0.0
        )
        z2 = jnp.dot(a, w2_ref[...], preferred_element_type=jnp.float32) + b2_ref[...]
        mu = jnp.sum(z2, axis=1, keepdims=True) * (1.0 / H)
        var = jnp.sum(z2 * z2, axis=1, keepdims=True) * (1.0 / H) - mu * mu
        zn = (z2 - mu) * lax.rsqrt(var + 1e-5) * g_ref[...] + bb_ref[...]
        h2_ref[pl.ds(i * ROWS_BLK, ROWS_BLK), :] = h + jnp.maximum(zn, 0.0)

    @pl.when(i == N // ROWS_BLK)
    def _():
        _pool_body(h2_ref, bcol_ref, brow_ref, gf_ref, wg1_ref, bg1_ref,
                   wg2_ref, bg2_ref, wca_ref, wcb_ref, bc1_ref, wc2_ref,
                   bc2_ref, o_ref)


def _full(shape):
    return pl.BlockSpec(shape, lambda *_: tuple(0 for _ in shape))


def _h0_call(x, s, embp):
    grid = (N // ROWS_BLK,)
    return pl.pallas_call(
        _h0_body,
        grid=grid,
        in_specs=[
            pl.BlockSpec((ROWS_BLK, F_IN), lambda i: (i, 0)),
            _full((F_IN, HP)),
            _full((NTYPES, HP)),
        ],
        out_specs=pl.BlockSpec((ROWS_BLK, HP), lambda i: (i, 0)),
        out_shape=jax.ShapeDtypeStruct((N, HP), jnp.float32),
    )(x, s, embp)


def _mlp_call(h, p, w1, b1, w2, b2, g, bb):
    grid = (N // ROWS_BLK,)
    return pl.pallas_call(
        _mlp_body,
        grid=grid,
        in_specs=[
            pl.BlockSpec((ROWS_BLK, HP), lambda i: (i, 0)),
            pl.BlockSpec((2, ROWS_BLK, HP), lambda i: (0, i, 0)),
            _full((HP, HP)),
            _full((1, HP)),
            _full((HP, HP)),
            _full((1, HP)),
            _full((1, HP)),
            _full((1, HP)),
        ],
        out_specs=pl.BlockSpec((ROWS_BLK, HP), lambda i: (i, 0)),
        out_shape=jax.ShapeDtypeStruct((N, HP), jnp.float32),
    )(h, p, w1, b1, w2, b2, g, bb)


def _mlp_pool_call(h, p, w1, b1, w2, b2, g, bb,
                   bcol, brow, gf, wg1, bg1, wg2, bg2, wca, wcb, bc1, wc2, bc2):
    nblk = N // ROWS_BLK
    clamp = lambda i: (jnp.minimum(i, nblk - 1), 0)
    clamp3 = lambda i: (0, jnp.minimum(i, nblk - 1), 0)
    return pl.pallas_call(
        _mlp_pool_body,
        grid=(nblk + 1,),
        in_specs=[
            pl.BlockSpec((ROWS_BLK, HP), clamp),
            pl.BlockSpec((2, ROWS_BLK, HP), clamp3),
            _full((HP, HP)),
            _full((1, HP)),
            _full((HP, HP)),
            _full((1, HP)),
            _full((1, HP)),
            _full((1, HP)),
            _full((N, 1)),
            _full((1, N)),
            _full((G, GF)),
            _full((HP, PH)),
            _full((1, PH)),
            _full((1, PH)),
            _full((1, 1)),
            _full((HP, PH)),
            _full((GF, PH)),
            _full((1, PH)),
            _full((PH, 2)),
            _full((1, 2)),
        ],
        out_specs=_full((G, 2)),
        out_shape=jax.ShapeDtypeStruct((G, 2), jnp.float32),
        scratch_shapes=[pltpu.VMEM((N, HP), jnp.float32)],
    )(h, p, w1, b1, w2, b2, g, bb, bcol, brow, gf, wg1, bg1, wg2, bg2,
      wca, wcb, bc1, wc2, bc2)


def _sc_agg_body(h_hbm, src_hbm, dst_hbm, zeros_hbm, out_hbm,
                 src_v, dst_v, rows0, rows1, rows2, rows3, acc_sh,
                 sem0, sem1, sem2, sem3):
    rows = (rows0, rows1, rows2, rows3)
    sems = (sem0, sem1, sem2, sem3)
    c = lax.axis_index("c")
    s = lax.axis_index("s")
    w = c * NSUB + s
    pltpu.sync_copy(zeros_hbm, acc_sh.at[pl.ds(s * RPT, RPT)])
    plsc.subcore_barrier()

    def block(bb, carry):
        base = w * NCHUNK + bb * IB
        pltpu.sync_copy(src_hbm.at[pl.ds(base, IB)], src_v)
        pltpu.sync_copy(dst_hbm.at[pl.ds(base, IB)], dst_v)
        for b in range(NBUF - 1):
            pltpu.async_copy(h_hbm.at[src_v.at[b]], rows[b], sems[b])

        def group(jj, carry2):
            j0 = jj * NBUF
            for b in range(NBUF):
                j = j0 + b
                pltpu.make_async_copy(
                    h_hbm.at[src_v.at[j]], rows[b], sems[b]
                ).wait()
                nxt = j + NBUF - 1
                bn = (b + NBUF - 1) % NBUF

                @pl.when(nxt < IB)
                def _():
                    pltpu.async_copy(
                        h_hbm.at[src_v.at[nxt]], rows[bn], sems[bn]
                    )

                pltpu.sync_copy(rows[b], acc_sh.at[dst_v.at[j]], add=True)
            return carry2

        lax.fori_loop(0, IB // NBUF, group, 0)
        return carry

    lax.fori_loop(0, NB, block, 0)
    plsc.subcore_barrier()
    pltpu.sync_copy(
        acc_sh.at[pl.ds(s * RPT, RPT)],
        out_hbm.at[pl.ds(c * N + s * RPT, RPT)],
    )


@functools.cache
def _sc_agg_kernel():
    mesh = plsc.VectorSubcoreMesh(core_axis_name="c", subcore_axis_name="s")
    return pl.kernel(
        _sc_agg_body,
        out_type=jax.ShapeDtypeStruct((2 * N, HP), jnp.float32),
        mesh=mesh,
        compiler_params=pltpu.CompilerParams(use_tc_tiling_on_sc=False),
        scratch_types=[
            pltpu.VMEM((IB, CHUNK), jnp.int32),
            pltpu.VMEM((IB, CHUNK), jnp.int32),
            pltpu.VMEM((CHUNK, HP), jnp.float32),
            pltpu.VMEM((CHUNK, HP), jnp.float32),
            pltpu.VMEM((CHUNK, HP), jnp.float32),
            pltpu.VMEM((CHUNK, HP), jnp.float32),
            pltpu.VMEM_SHARED((N, HP), jnp.float32),
            pltpu.SemaphoreType.DMA,
            pltpu.SemaphoreType.DMA,
            pltpu.SemaphoreType.DMA,
            pltpu.SemaphoreType.DMA,
        ],
    )


def kernel(x, edge_index, batch, global_features, emb, W1a, b1a, W2a, b2a,
           W1b, b1b, W2b, b2b, ln1g, ln1b, ln2g, ln2b, Wg1, bg1, Wg2, bg2,
           Wc1, bc1, Wc2, bc2):
    f32 = jnp.float32

    s_mat = jnp.eye(F_IN, HP, k=-1, dtype=f32)
    embp = jnp.pad(emb, ((0, 0), (F_IN - 1, HP - F_IN + 1 - EMB)))

    def pad2(wm):
        return jnp.pad(wm, ((0, HP - H), (0, HP - H)))

    def padr(v):
        return jnp.pad(v, (0, HP - H)).reshape(1, HP)

    w1a, w2a, w1b, w2b = pad2(W1a), pad2(W2a), pad2(W1b), pad2(W2b)
    b1a_, b2a_, b1b_, b2b_ = padr(b1a), padr(b2a), padr(b1b), padr(b2b)
    g1, bb1, g2, bb2 = padr(ln1g), padr(ln1b), padr(ln2g), padr(ln2b)
    wg1 = jnp.pad(Wg1, ((0, HP - H), (0, 0)))
    wg2 = Wg2.reshape(1, PH)
    bg1_ = bg1.reshape(1, PH)
    bg2_ = bg2.reshape(1, 1)
    wca = jnp.pad(Wc1[:H], ((0, HP - H), (0, 0)))
    wcb = Wc1[H:]
    bc1_ = bc1.reshape(1, PH)
    bc2_ = bc2.reshape(1, 2)

    src3 = edge_index[0].reshape(NW * NCHUNK, CHUNK)
    dst3 = edge_index[1].reshape(NW * NCHUNK, CHUNK)
    zeros_chunk = jnp.zeros((RPT, HP), f32)
    bcol = batch.reshape(N, 1)
    brow = batch.reshape(1, N)

    sc_agg = _sc_agg_kernel()

    h0 = _h0_call(x, s_mat, embp)
    p1 = sc_agg(h0, src3, dst3, zeros_chunk).reshape(2, N, HP)
    h1 = _mlp_call(h0, p1, w1a, b1a_, w2a, b2a_, g1, bb1)
    p2 = sc_agg(h1, src3, dst3, zeros_chunk).reshape(2, N, HP)
    return _mlp_pool_call(h1, p2, w1b, b1b_, w2b, b2b_, g2, bb2,
                          bcol, brow, global_features, wg1, bg1_, wg2, bg2_,
                          wca, wcb, bc1_, Wc2, bc2_)


# speedup vs baseline: 1.0974x; 1.0342x over previous
import functools

import jax
import jax.numpy as jnp
from jax import lax
from jax.experimental import pallas as pl
from jax.experimental.pallas import tpu as pltpu
from jax.experimental.pallas import tpu_sc as plsc

N = 10000
E = 320000
F_IN = 128
EMB = 16
H = 143
HP = 144
G = 64
GF = 32
PH = 128
NTYPES = 400

NW = 32
EPW = E // NW
CHUNK = 50
NCHUNK = EPW // CHUNK
IB = 100
NBUF = 4
NB = NCHUNK // IB
NSUB = 16
RPT = N // NSUB

ROWS_BLK = 2000


def _h0_body(x_ref, s_ref, embp_ref, o_ref):
    x = x_ref[...]
    nt = x[:, 0:1].astype(jnp.int32)
    iota = lax.broadcasted_iota(jnp.int32, (1, NTYPES), 1)
    oh = (nt == iota).astype(jnp.float32)
    o_ref[...] = (
        jnp.dot(x, s_ref[...], preferred_element_type=jnp.float32)
        + jnp.dot(oh, embp_ref[...], preferred_element_type=jnp.float32)
    )


def _mlp_body(h_ref, p_ref, w1_ref, b1_ref, w2_ref, b2_ref, g_ref, bb_ref, o_ref):
    h = h_ref[...]
    z = h + p_ref[0] + p_ref[1]
    a = jnp.maximum(
        jnp.dot(z, w1_ref[...], preferred_element_type=jnp.float32) + b1_ref[...], 0.0
    )
    z2 = jnp.dot(a, w2_ref[...], preferred_element_type=jnp.float32) + b2_ref[...]
    mu = jnp.sum(z2, axis=1, keepdims=True) * (1.0 / H)
    var = jnp.sum(z2 * z2, axis=1, keepdims=True) * (1.0 / H) - mu * mu
    zn = (z2 - mu) * lax.rsqrt(var + 1e-5) * g_ref[...] + bb_ref[...]
    o_ref[...] = h + jnp.maximum(zn, 0.0)


def _pool_body(h_ref, bcol_ref, brow_ref, gf_ref, wg1_ref, bg1_ref, wg2_ref,
               bg2_ref, wca_ref, wcb_ref, bc1_ref, wc2_ref, bc2_ref, o_ref):
    h = h_ref[...]
    a = jnp.maximum(
        jnp.dot(h, wg1_ref[...], preferred_element_type=jnp.float32) + bg1_ref[...], 0.0
    )
    gate = jnp.sum(a * wg2_ref[...], axis=1, keepdims=True) + bg2_ref[...]
    bcol = bcol_ref[...]
    ohb = bcol == lax.broadcasted_iota(jnp.int32, (1, G), 1)
    ohf = ohb.astype(jnp.float32)
    gmax = jnp.max(jnp.where(ohb, gate, -3e38), axis=0, keepdims=True)
    gmax_pn = jnp.sum(ohf * gmax, axis=1, keepdims=True)
    e = jnp.exp(gate - gmax_pn)
    denom = jnp.sum(ohf * e, axis=0, keepdims=True)
    denom_pn = jnp.sum(ohf * denom, axis=1, keepdims=True)
    alpha = e / (denom_pn + 1e-16)
    oht = (brow_ref[...] == lax.broadcasted_iota(jnp.int32, (G, 1), 0)).astype(
        jnp.float32
    )
    pooled = jnp.dot(oht, alpha * h, preferred_element_type=jnp.float32)
    c1 = jnp.maximum(
        jnp.dot(pooled, wca_ref[...], preferred_element_type=jnp.float32)
        + jnp.dot(gf_ref[...], wcb_ref[...], preferred_element_type=jnp.float32)
        + bc1_ref[...],
        0.0,
    )
    o_ref[...] = jnp.dot(c1, wc2_ref[...], preferred_element_type=jnp.float32) + bc2_ref[...]


def _mlp_pool_body(h_ref, p_ref, w1_ref, b1_ref, w2_ref, b2_ref, g_ref, bb_ref,
                   bcol_ref, brow_ref, gf_ref, wg1_ref, bg1_ref, wg2_ref,
                   bg2_ref, wca_ref, wcb_ref, bc1_ref, wc2_ref, bc2_ref,
                   o_ref, h2_ref):
    i = pl.program_id(0)

    @pl.when(i < N // ROWS_BLK)
    def _():
        h = h_ref[...]
        z = h + p_ref[0] + p_ref[1]
        a = jnp.maximum(
            jnp.dot(z, w1_ref[...], preferred_element_type=jnp.float32)
            + b1_ref[...], 0.0
        )
        z2 = jnp.dot(a, w2_ref[...], preferred_element_type=jnp.float32) + b2_ref[...]
        mu = jnp.sum(z2, axis=1, keepdims=True) * (1.0 / H)
        var = jnp.sum(z2 * z2, axis=1, keepdims=True) * (1.0 / H) - mu * mu
        zn = (z2 - mu) * lax.rsqrt(var + 1e-5) * g_ref[...] + bb_ref[...]
        h2_ref[pl.ds(i * ROWS_BLK, ROWS_BLK), :] = h + jnp.maximum(zn, 0.0)

    @pl.when(i == N // ROWS_BLK)
    def _():
        _pool_body(h2_ref, bcol_ref, brow_ref, gf_ref, wg1_ref, bg1_ref,
                   wg2_ref, bg2_ref, wca_ref, wcb_ref, bc1_ref, wc2_ref,
                   bc2_ref, o_ref)


def _full(shape):
    return pl.BlockSpec(shape, lambda *_: tuple(0 for _ in shape))


def _h0_call(x, s, embp):
    grid = (N // ROWS_BLK,)
    return pl.pallas_call(
        _h0_body,
        grid=grid,
        in_specs=[
            pl.BlockSpec((ROWS_BLK, F_IN), lambda i: (i, 0)),
            _full((F_IN, HP)),
            _full((NTYPES, HP)),
        ],
        out_specs=pl.BlockSpec((ROWS_BLK, HP), lambda i: (i, 0)),
        out_shape=jax.ShapeDtypeStruct((N, HP), jnp.float32),
    )(x, s, embp)


def _mlp_call(h, p, w1, b1, w2, b2, g, bb):
    grid = (N // ROWS_BLK,)
    return pl.pallas_call(
        _mlp_body,
        grid=grid,
        in_specs=[
            pl.BlockSpec((ROWS_BLK, HP), lambda i: (i, 0)),
            pl.BlockSpec((2, ROWS_BLK, HP), lambda i: (0, i, 0)),
            _full((HP, HP)),
            _full((1, HP)),
            _full((HP, HP)),
            _full((1, HP)),
            _full((1, HP)),
            _full((1, HP)),
        ],
        out_specs=pl.BlockSpec((ROWS_BLK, HP), lambda i: (i, 0)),
        out_shape=jax.ShapeDtypeStruct((N, HP), jnp.float32),
    )(h, p, w1, b1, w2, b2, g, bb)


def _mlp_pool_call(h, p, w1, b1, w2, b2, g, bb,
                   bcol, brow, gf, wg1, bg1, wg2, bg2, wca, wcb, bc1, wc2, bc2):
    nblk = N // ROWS_BLK
    clamp = lambda i: (jnp.minimum(i, nblk - 1), 0)
    clamp3 = lambda i: (0, jnp.minimum(i, nblk - 1), 0)
    return pl.pallas_call(
        _mlp_pool_body,
        grid=(nblk + 1,),
        in_specs=[
            pl.BlockSpec((ROWS_BLK, HP), clamp),
            pl.BlockSpec((2, ROWS_BLK, HP), clamp3),
            _full((HP, HP)),
            _full((1, HP)),
            _full((HP, HP)),
            _full((1, HP)),
            _full((1, HP)),
            _full((1, HP)),
            _full((N, 1)),
            _full((1, N)),
            _full((G, GF)),
            _full((HP, PH)),
            _full((1, PH)),
            _full((1, PH)),
            _full((1, 1)),
            _full((HP, PH)),
            _full((GF, PH)),
            _full((1, PH)),
            _full((PH, 2)),
            _full((1, 2)),
        ],
        out_specs=_full((G, 2)),
        out_shape=jax.ShapeDtypeStruct((G, 2), jnp.float32),
        scratch_shapes=[pltpu.VMEM((N, HP), jnp.float32)],
    )(h, p, w1, b1, w2, b2, g, bb, bcol, brow, gf, wg1, bg1, wg2, bg2,
      wca, wcb, bc1, wc2, bc2)


def _sc_agg_body(h_hbm, src_hbm, dst_hbm, zeros_hbm, out_hbm,
                 src_v, dst_v, rows0, rows1, rows2, rows3, acc_sh,
                 sem0, sem1, sem2, sem3):
    rows = (rows0, rows1, rows2, rows3)
    sems = (sem0, sem1, sem2, sem3)
    c = lax.axis_index("c")
    s = lax.axis_index("s")
    w = c * NSUB + s
    pltpu.sync_copy(zeros_hbm, acc_sh.at[pl.ds(s * RPT, RPT)])
    plsc.subcore_barrier()

    def block(bb, carry):
        base = w * NCHUNK + bb * IB
        pltpu.sync_copy(src_hbm.at[pl.ds(base, IB)], src_v)
        pltpu.sync_copy(dst_hbm.at[pl.ds(base, IB)], dst_v)
        for b in range(NBUF - 1):
            pltpu.async_copy(h_hbm.at[src_v.at[b]], rows[b], sems[b])

        def group(jj, carry2):
            j0 = jj * NBUF
            for b in range(NBUF):
                j = j0 + b
                pltpu.make_async_copy(
                    h_hbm.at[src_v.at[j]], rows[b], sems[b]
                ).wait()
                nxt = j + NBUF - 1
                bn = (b + NBUF - 1) % NBUF

                @pl.when(nxt < IB)
                def _():
                    pltpu.async_copy(
                        h_hbm.at[src_v.at[nxt]], rows[bn], sems[bn]
                    )

                pltpu.sync_copy(rows[b], acc_sh.at[dst_v.at[j]], add=True)
            return carry2

        lax.fori_loop(0, IB // NBUF, group, 0)
        return carry

    lax.fori_loop(0, NB, block, 0)
    plsc.subcore_barrier()
    pltpu.sync_copy(
        acc_sh.at[pl.ds(s * RPT, RPT)],
        out_hbm.at[pl.ds(c * N + s * RPT, RPT)],
    )


@functools.cache
def _sc_agg_kernel():
    mesh = plsc.VectorSubcoreMesh(core_axis_name="c", subcore_axis_name="s")
    return pl.kernel(
        _sc_agg_body,
        out_type=jax.ShapeDtypeStruct((2 * N, HP), jnp.float32),
        mesh=mesh,
        compiler_params=pltpu.CompilerParams(use_tc_tiling_on_sc=False),
        scratch_types=[
            pltpu.VMEM((IB, CHUNK), jnp.int32),
            pltpu.VMEM((IB, CHUNK), jnp.int32),
            pltpu.VMEM((CHUNK, HP), jnp.float32),
            pltpu.VMEM((CHUNK, HP), jnp.float32),
            pltpu.VMEM((CHUNK, HP), jnp.float32),
            pltpu.VMEM((CHUNK, HP), jnp.float32),
            pltpu.VMEM_SHARED((N, HP), jnp.float32),
            pltpu.SemaphoreType.DMA,
            pltpu.SemaphoreType.DMA,
            pltpu.SemaphoreType.DMA,
            pltpu.SemaphoreType.DMA,
        ],
    )


def kernel(x, edge_index, batch, global_features, emb, W1a, b1a, W2a, b2a,
           W1b, b1b, W2b, b2b, ln1g, ln1b, ln2g, ln2b, Wg1, bg1, Wg2, bg2,
           Wc1, bc1, Wc2, bc2):
    f32 = jnp.float32

    s_mat = jnp.eye(F_IN, HP, k=-1, dtype=f32)
    embp = jnp.pad(emb, ((0, 0), (F_IN - 1, HP - F_IN + 1 - EMB)))

    def pad2(wm):
        return jnp.pad(wm, ((0, HP - H), (0, HP - H)))

    def padr(v):
        return jnp.pad(v, (0, HP - H)).reshape(1, HP)

    w1a, w2a, w1b, w2b = pad2(W1a), pad2(W2a), pad2(W1b), pad2(W2b)
    b1a_, b2a_, b1b_, b2b_ = padr(b1a), padr(b2a), padr(b1b), padr(b2b)
    g1, bb1, g2, bb2 = padr(ln1g), padr(ln1b), padr(ln2g), padr(ln2b)
    wg1 = jnp.pad(Wg1, ((0, HP - H), (0, 0)))
    wg2 = Wg2.reshape(1, PH)
    bg1_ = bg1.reshape(1, PH)
    bg2_ = bg2.reshape(1, 1)
    wca = jnp.pad(Wc1[:H], ((0, HP - H), (0, 0)))
    wcb = Wc1[H:]
    bc1_ = bc1.reshape(1, PH)
    bc2_ = bc2.reshape(1, 2)

    src3 = edge_index[0].reshape(NW * NCHUNK, CHUNK)
    dst3 = edge_index[1].reshape(NW * NCHUNK, CHUNK)
    zeros_chunk = jnp.zeros((RPT, HP), f32)
    bcol = batch.reshape(N, 1)
    brow = batch.reshape(1, N)

    sc_agg = _sc_agg_kernel()

    h0 = _h0_call(x, s_mat, embp)
    p1 = sc_agg(h0, src3, dst3, zeros_chunk).reshape(2, N, HP)
    h1 = _mlp_call(h0, p1, w1a, b1a_, w2a, b2a_, g1, bb1)
    p2 = sc_agg(h1, src3, dst3, zeros_chunk).reshape(2, N, HP)
    return _mlp_pool_call(h1, p2, w1b, b1b_, w2b, b2b_, g2, bb2,
                          bcol, brow, global_features, wg1, bg1_, wg2, bg2_,
                          wca, wcb, bc1_, Wc2, bc2_)
